# baseline (speedup 1.0000x reference)
import functools

import jax
import jax.numpy as jnp
from jax.experimental import pallas as pl
from jax.experimental.pallas import tpu as pltpu

EMBED = 768
NEXP = 4
NCLS = 8
ALPHA = 0.01
ROWS_PER_BLOCK = 1024

NEG_INF = float("-inf")


def _gate_kernel(x_ref, wt_ref, vals_ref, idx_ref, aux_ref, acc_ref):
    i = pl.program_id(0)
    n = pl.num_programs(0)

    @pl.when(i == 0)
    def _():
        acc_ref[0] = 0.0
        acc_ref[1] = 0.0

    x = x_ref[...]
    wt = wt_ref[...]
    logits = jnp.dot(x, wt, preferred_element_type=jnp.float32)

    m = jnp.max(logits, axis=1, keepdims=True)
    e = jnp.exp(logits - m)
    s = jnp.sum(e, axis=1, keepdims=True)
    p = e / s

    p0 = p[:, 0:1]
    p1 = p[:, 1:2]
    p2 = p[:, 2:3]
    p3 = p[:, 3:4]

    v1 = jnp.maximum(jnp.maximum(p0, p1), jnp.maximum(p2, p3))
    i1 = jnp.where(p0 == v1, 0,
         jnp.where(p1 == v1, 1,
         jnp.where(p2 == v1, 2, 3))).astype(jnp.int32)

    q0 = jnp.where(i1 == 0, NEG_INF, p0)
    q1 = jnp.where(i1 == 1, NEG_INF, p1)
    q2 = jnp.where(i1 == 2, NEG_INF, p2)
    q3 = jnp.where(i1 == 3, NEG_INF, p3)
    v2 = jnp.maximum(jnp.maximum(q0, q1), jnp.maximum(q2, q3))
    i2 = jnp.where(q0 == v2, 0,
         jnp.where(q1 == v2, 1,
         jnp.where(q2 == v2, 2, 3))).astype(jnp.int32)

    vals_ref[...] = jnp.concatenate([v1, v2], axis=1)
    idx_ref[...] = jnp.concatenate([i1, i2], axis=1)

    acc_ref[0] += jnp.sum(p)
    hits = ((v1 == 0.0) | (v1 == 1.0)).astype(jnp.float32) + \
           ((v2 == 0.0) | (v2 == 1.0)).astype(jnp.float32)
    acc_ref[1] += jnp.sum(hits)

    @pl.when(i == n - 1)
    def _():
        total = jnp.float32(n * ROWS_PER_BLOCK)
        pi = acc_ref[0] / (total * NEXP)
        ce_sum = acc_ref[1] / (total * 2)
        aux_ref[0, 0] = pi * ce_sum * jnp.float32(NCLS) * jnp.float32(ALPHA)


@jax.jit
def kernel(hidden_states, weight):
    b, t, h = hidden_states.shape
    ntok = b * t
    hs = jnp.reshape(hidden_states, (ntok, h))
    wt = weight.T
    grid = ntok // ROWS_PER_BLOCK

    vals, idx, aux = pl.pallas_call(
        _gate_kernel,
        grid=(grid,),
        in_specs=[
            pl.BlockSpec((ROWS_PER_BLOCK, h), lambda i: (i, 0)),
            pl.BlockSpec((h, NEXP), lambda i: (0, 0)),
        ],
        out_specs=[
            pl.BlockSpec((ROWS_PER_BLOCK, 2), lambda i: (i, 0)),
            pl.BlockSpec((ROWS_PER_BLOCK, 2), lambda i: (i, 0)),
            pl.BlockSpec(memory_space=pltpu.SMEM),
        ],
        out_shape=[
            jax.ShapeDtypeStruct((ntok, 2), jnp.float32),
            jax.ShapeDtypeStruct((ntok, 2), jnp.int32),
            jax.ShapeDtypeStruct((1, 1), jnp.float32),
        ],
        scratch_shapes=[pltpu.SMEM((2,), jnp.float32)],
    )(hs, wt)

    return (vals, idx, jnp.reshape(aux, ()))


# speedup vs baseline: 2.9026x; 2.9026x over previous
import jax
import jax.numpy as jnp
from jax.experimental import pallas as pl
from jax.experimental.pallas import tpu as pltpu

EMBED = 768
NEXP = 4
NCLS = 8
ALPHA = 0.01
ROWS_PER_BLOCK = 1024

NEG_INF = float("-inf")


def _gate_kernel(x_ref, w_ref, vals_ref, idx_ref, aux_ref, acc_ref):
    i = pl.program_id(0)
    n = pl.num_programs(0)

    @pl.when(i == 0)
    def _():
        acc_ref[0] = 0.0
        acc_ref[1] = 0.0

    x = x_ref[...]
    w = w_ref[...]
    lt = jax.lax.dot_general(w, x, (((1,), (1,)), ((), ())),
                             preferred_element_type=jnp.float32)

    m = jnp.max(lt, axis=0, keepdims=True)
    e = jnp.exp(lt - m)
    s = jnp.sum(e, axis=0, keepdims=True)
    p = e / s

    p0 = p[0:1, :]
    p1 = p[1:2, :]
    p2 = p[2:3, :]
    p3 = p[3:4, :]

    v1 = jnp.maximum(jnp.maximum(p0, p1), jnp.maximum(p2, p3))
    i1 = jnp.where(p0 == v1, 0,
         jnp.where(p1 == v1, 1,
         jnp.where(p2 == v1, 2, 3))).astype(jnp.int32)

    q0 = jnp.where(i1 == 0, NEG_INF, p0)
    q1 = jnp.where(i1 == 1, NEG_INF, p1)
    q2 = jnp.where(i1 == 2, NEG_INF, p2)
    q3 = jnp.where(i1 == 3, NEG_INF, p3)
    v2 = jnp.maximum(jnp.maximum(q0, q1), jnp.maximum(q2, q3))
    i2 = jnp.where(q0 == v2, 0,
         jnp.where(q1 == v2, 1,
         jnp.where(q2 == v2, 2, 3))).astype(jnp.int32)

    vals_ref[...] = jnp.concatenate([v1, v2], axis=0)
    idx_ref[...] = jnp.concatenate([i1, i2], axis=0)

    acc_ref[0] += jnp.sum(p)
    hits = ((v1 == 0.0) | (v1 == 1.0)).astype(jnp.float32) + \
           ((v2 == 0.0) | (v2 == 1.0)).astype(jnp.float32)
    acc_ref[1] += jnp.sum(hits)

    @pl.when(i == n - 1)
    def _():
        total = jnp.float32(n * ROWS_PER_BLOCK)
        pi = acc_ref[0] / (total * NEXP)
        ce_sum = acc_ref[1] / (total * 2)
        aux_ref[0, 0] = pi * ce_sum * jnp.float32(NCLS) * jnp.float32(ALPHA)


@jax.jit
def kernel(hidden_states, weight):
    b, t, h = hidden_states.shape
    ntok = b * t
    hs = jnp.reshape(hidden_states, (ntok, h))
    grid = ntok // ROWS_PER_BLOCK

    vals_t, idx_t, aux = pl.pallas_call(
        _gate_kernel,
        grid=(grid,),
        in_specs=[
            pl.BlockSpec((ROWS_PER_BLOCK, h), lambda i: (i, 0)),
            pl.BlockSpec((NEXP, h), lambda i: (0, 0)),
        ],
        out_specs=[
            pl.BlockSpec((2, ROWS_PER_BLOCK), lambda i: (0, i)),
            pl.BlockSpec((2, ROWS_PER_BLOCK), lambda i: (0, i)),
            pl.BlockSpec(memory_space=pltpu.SMEM),
        ],
        out_shape=[
            jax.ShapeDtypeStruct((2, ntok), jnp.float32),
            jax.ShapeDtypeStruct((2, ntok), jnp.int32),
            jax.ShapeDtypeStruct((1, 1), jnp.float32),
        ],
        scratch_shapes=[pltpu.SMEM((2,), jnp.float32)],
    )(hs, weight)

    return (vals_t.T, idx_t.T, jnp.reshape(aux, ()))


# speedup vs baseline: 3.7124x; 1.2790x over previous
import jax
import jax.numpy as jnp
from jax.experimental import pallas as pl
from jax.experimental.pallas import tpu as pltpu

EMBED = 768
NEXP = 4
NCLS = 8
ALPHA = 0.01
ROWS_PER_BLOCK = 2048

NEG_INF = float("-inf")


def _gate_kernel(x_ref, w_ref, vals_ref, idx_ref, aux_ref, acc_ref):
    i = pl.program_id(0)
    n = pl.num_programs(0)

    @pl.when(i == 0)
    def _():
        acc_ref[0] = 0.0
        acc_ref[1] = 0.0

    x = x_ref[...]
    w = w_ref[...]
    lt = jax.lax.dot_general(w, x, (((1,), (1,)), ((), ())),
                             preferred_element_type=jnp.float32)

    m = jnp.max(lt, axis=0, keepdims=True)
    e = jnp.exp(lt - m)
    s = jnp.sum(e, axis=0, keepdims=True)
    p = e / s

    p0 = p[0:1, :]
    p1 = p[1:2, :]
    p2 = p[2:3, :]
    p3 = p[3:4, :]

    v1 = jnp.maximum(jnp.maximum(p0, p1), jnp.maximum(p2, p3))
    i1 = jnp.where(p0 == v1, 0,
         jnp.where(p1 == v1, 1,
         jnp.where(p2 == v1, 2, 3))).astype(jnp.int32)

    q0 = jnp.where(i1 == 0, NEG_INF, p0)
    q1 = jnp.where(i1 == 1, NEG_INF, p1)
    q2 = jnp.where(i1 == 2, NEG_INF, p2)
    q3 = jnp.where(i1 == 3, NEG_INF, p3)
    v2 = jnp.maximum(jnp.maximum(q0, q1), jnp.maximum(q2, q3))
    i2 = jnp.where(q0 == v2, 0,
         jnp.where(q1 == v2, 1,
         jnp.where(q2 == v2, 2, 3))).astype(jnp.int32)

    vals_ref[...] = jnp.concatenate([v1, v2], axis=0)
    idx_ref[...] = jnp.concatenate([i1, i2], axis=0)

    acc_ref[0] += jnp.sum(p)
    hits = ((v1 == 0.0) | (v1 == 1.0)).astype(jnp.float32) + \
           ((v2 == 0.0) | (v2 == 1.0)).astype(jnp.float32)
    acc_ref[1] += jnp.sum(hits)

    @pl.when(i == n - 1)
    def _():
        total = jnp.float32(n * ROWS_PER_BLOCK)
        pi = acc_ref[0] / (total * NEXP)
        ce_sum = acc_ref[1] / (total * 2)
        aux_ref[0, 0] = pi * ce_sum * jnp.float32(NCLS) * jnp.float32(ALPHA)


@jax.jit
def kernel(hidden_states, weight):
    b, t, h = hidden_states.shape
    ntok = b * t
    hs = jnp.reshape(hidden_states, (ntok, h))
    grid = ntok // ROWS_PER_BLOCK

    vals_t, idx_t, aux = pl.pallas_call(
        _gate_kernel,
        grid=(grid,),
        in_specs=[
            pl.BlockSpec((ROWS_PER_BLOCK, h), lambda i: (i, 0)),
            pl.BlockSpec((NEXP, h), lambda i: (0, 0)),
        ],
        out_specs=[
            pl.BlockSpec((2, ROWS_PER_BLOCK), lambda i: (0, i)),
            pl.BlockSpec((2, ROWS_PER_BLOCK), lambda i: (0, i)),
            pl.BlockSpec(memory_space=pltpu.SMEM),
        ],
        out_shape=[
            jax.ShapeDtypeStruct((2, ntok), jnp.float32),
            jax.ShapeDtypeStruct((2, ntok), jnp.int32),
            jax.ShapeDtypeStruct((1, 1), jnp.float32),
        ],
        scratch_shapes=[pltpu.SMEM((2,), jnp.float32)],
    )(hs, weight)

    return (vals_t.T, idx_t.T, jnp.reshape(aux, ()))


# speedup vs baseline: 3.8992x; 1.0503x over previous
import jax
import jax.numpy as jnp
from jax.experimental import pallas as pl
from jax.experimental.pallas import tpu as pltpu

EMBED = 768
NEXP = 4
NCLS = 8
ALPHA = 0.01
ROWS_PER_BLOCK = 4096

NEG_INF = float("-inf")


def _gate_kernel(x_ref, w_ref, vals_ref, idx_ref, aux_ref, acc_ref):
    i = pl.program_id(0)
    n = pl.num_programs(0)

    @pl.when(i == 0)
    def _():
        acc_ref[0] = 0.0
        acc_ref[1] = 0.0

    x = x_ref[...]
    w = w_ref[...]
    lt = jax.lax.dot_general(w, x, (((1,), (1,)), ((), ())),
                             preferred_element_type=jnp.float32)

    m = jnp.max(lt, axis=0, keepdims=True)
    e = jnp.exp(lt - m)
    s = jnp.sum(e, axis=0, keepdims=True)
    p = e / s

    p0 = p[0:1, :]
    p1 = p[1:2, :]
    p2 = p[2:3, :]
    p3 = p[3:4, :]

    v1 = jnp.maximum(jnp.maximum(p0, p1), jnp.maximum(p2, p3))
    i1 = jnp.where(p0 == v1, 0,
         jnp.where(p1 == v1, 1,
         jnp.where(p2 == v1, 2, 3))).astype(jnp.int32)

    q0 = jnp.where(i1 == 0, NEG_INF, p0)
    q1 = jnp.where(i1 == 1, NEG_INF, p1)
    q2 = jnp.where(i1 == 2, NEG_INF, p2)
    q3 = jnp.where(i1 == 3, NEG_INF, p3)
    v2 = jnp.maximum(jnp.maximum(q0, q1), jnp.maximum(q2, q3))
    i2 = jnp.where(q0 == v2, 0,
         jnp.where(q1 == v2, 1,
         jnp.where(q2 == v2, 2, 3))).astype(jnp.int32)

    vals_ref[...] = jnp.concatenate([v1, v2], axis=0)
    idx_ref[...] = jnp.concatenate([i1, i2], axis=0)

    acc_ref[0] += jnp.sum(p)
    hits = ((v1 == 0.0) | (v1 == 1.0)).astype(jnp.float32) + \
           ((v2 == 0.0) | (v2 == 1.0)).astype(jnp.float32)
    acc_ref[1] += jnp.sum(hits)

    @pl.when(i == n - 1)
    def _():
        total = jnp.float32(n * ROWS_PER_BLOCK)
        pi = acc_ref[0] / (total * NEXP)
        ce_sum = acc_ref[1] / (total * 2)
        aux_ref[0, 0] = pi * ce_sum * jnp.float32(NCLS) * jnp.float32(ALPHA)


@jax.jit
def kernel(hidden_states, weight):
    b, t, h = hidden_states.shape
    ntok = b * t
    hs = jnp.reshape(hidden_states, (ntok, h))
    grid = ntok // ROWS_PER_BLOCK

    vals_t, idx_t, aux = pl.pallas_call(
        _gate_kernel,
        grid=(grid,),
        in_specs=[
            pl.BlockSpec((ROWS_PER_BLOCK, h), lambda i: (i, 0)),
            pl.BlockSpec((NEXP, h), lambda i: (0, 0)),
        ],
        out_specs=[
            pl.BlockSpec((2, ROWS_PER_BLOCK), lambda i: (0, i)),
            pl.BlockSpec((2, ROWS_PER_BLOCK), lambda i: (0, i)),
            pl.BlockSpec(memory_space=pltpu.SMEM),
        ],
        out_shape=[
            jax.ShapeDtypeStruct((2, ntok), jnp.float32),
            jax.ShapeDtypeStruct((2, ntok), jnp.int32),
            jax.ShapeDtypeStruct((1, 1), jnp.float32),
        ],
        scratch_shapes=[pltpu.SMEM((2,), jnp.float32)],
    )(hs, weight)

    return (vals_t.T, idx_t.T, jnp.reshape(aux, ()))
